# speedup vs baseline: 2.5809x; 2.5809x over previous
import functools

import jax
import jax.numpy as jnp
from jax import lax
from jax.experimental import pallas as pl
from jax.experimental.pallas import tpu as pltpu
from jax.experimental.pallas import tpu_sc as plsc

L = 16
NUM_CORES = 2
NUM_SUBCORES = 16
NW = NUM_CORES * NUM_SUBCORES

CHUNK = 64


def _film_body(feats_hbm, idx_hbm, gamma_hbm, beta_hbm, out_hbm,
               idx_v, g_v, b_v, f_v, sem_g, sem_b, sem_f,
               *, rows_per_worker, n_chunks, d):
  wid = lax.axis_index("s") * NUM_CORES + lax.axis_index("c")
  base = wid * rows_per_worker

  def chunk_body(k, carry):
    off = base + k * CHUNK
    pltpu.sync_copy(idx_hbm.at[pl.ds(off, CHUNK)], idx_v)
    cg = pltpu.async_copy(gamma_hbm.at[idx_v], g_v, sem_g)
    cb = pltpu.async_copy(beta_hbm.at[idx_v], b_v, sem_b)
    cf = pltpu.async_copy(feats_hbm.at[pl.ds(off, CHUNK)], f_v, sem_f)
    cg.wait()
    cb.wait()
    cf.wait()

    def row_body(r, rcarry):
      for j in range(d // L):
        sl = pl.ds(j * L, L)
        f = f_v[r, sl]
        g = g_v[r, sl]
        b = b_v[r, sl]
        f_v[r, sl] = f + f * g + b
      return rcarry

    lax.fori_loop(0, CHUNK, row_body, 0)
    pltpu.sync_copy(f_v, out_hbm.at[pl.ds(off, CHUNK)])
    return carry

  lax.fori_loop(0, n_chunks, chunk_body, 0)


def kernel(feats, domain_idx, gamma, beta):
  n, d = feats.shape
  assert n % (NW * CHUNK) == 0 and d % L == 0
  rows_per_worker = n // NW
  n_chunks = rows_per_worker // CHUNK

  idx32 = domain_idx.astype(jnp.int32)

  mesh = plsc.VectorSubcoreMesh(core_axis_name="c", subcore_axis_name="s")
  body = functools.partial(
      _film_body, rows_per_worker=rows_per_worker, n_chunks=n_chunks, d=d)
  return pl.kernel(
      body,
      out_type=jax.ShapeDtypeStruct((n, d), jnp.float32),
      mesh=mesh,
      scratch_types=[
          pltpu.VMEM((CHUNK,), jnp.int32),
          pltpu.VMEM((CHUNK, d), jnp.float32),
          pltpu.VMEM((CHUNK, d), jnp.float32),
          pltpu.VMEM((CHUNK, d), jnp.float32),
          pltpu.SemaphoreType.DMA,
          pltpu.SemaphoreType.DMA,
          pltpu.SemaphoreType.DMA,
      ],
  )(feats, idx32, gamma, beta)


# speedup vs baseline: 3.1245x; 1.2106x over previous
import functools

import jax
import jax.numpy as jnp
from jax import lax
from jax.experimental import pallas as pl
from jax.experimental.pallas import tpu as pltpu
from jax.experimental.pallas import tpu_sc as plsc

L = 16
NUM_CORES = 2
NUM_SUBCORES = 16
NW = NUM_CORES * NUM_SUBCORES

CHUNK = 128


def _film_body(feats_hbm, idx_hbm, gamma_hbm, beta_hbm, out_hbm,
               idx_v, g0, b0, f0, g1, b1, f1,
               sem_idx, sem_in0, sem_in1, sem_out0, sem_out1,
               *, rows_per_worker, n_chunks, d):
  wid = lax.axis_index("s") * NUM_CORES + lax.axis_index("c")
  base = wid * rows_per_worker

  idx_cps = [
      pltpu.async_copy(idx_hbm.at[pl.ds(base + k * CHUNK, CHUNK)],
                       idx_v.at[k], sem_idx)
      for k in range(n_chunks)
  ]
  for cp in idx_cps:
    cp.wait()

  bufs = [(g0, b0, f0, sem_in0, sem_out0), (g1, b1, f1, sem_in1, sem_out1)]

  def start_in(k, g, b, f, sem):
    return [
        pltpu.async_copy(gamma_hbm.at[idx_v.at[k]], g, sem),
        pltpu.async_copy(beta_hbm.at[idx_v.at[k]], b, sem),
        pltpu.async_copy(feats_hbm.at[pl.ds(base + k * CHUNK, CHUNK)], f, sem),
    ]

  def compute(g, b, f):
    def row_body(r, rcarry):
      for j in range(d // L):
        sl = pl.ds(j * L, L)
        f[r, sl] = f[r, sl] + f[r, sl] * g[r, sl] + b[r, sl]
      return rcarry
    lax.fori_loop(0, CHUNK, row_body, 0)

  pending_in = [None, None]
  pending_out = [None, None]
  pending_in[0] = start_in(0, *bufs[0][:4])
  for k in range(n_chunks):
    s = k % 2
    ns = 1 - s
    if k + 1 < n_chunks:
      if pending_out[ns] is not None:
        pending_out[ns].wait()
      pending_in[ns] = start_in(k + 1, *bufs[ns][:4])
    for cp in pending_in[s]:
      cp.wait()
    g, b, f, _, sem_out = bufs[s]
    compute(g, b, f)
    pending_out[s] = pltpu.async_copy(
        f, out_hbm.at[pl.ds(base + k * CHUNK, CHUNK)], sem_out)
  for s in (0, 1):
    if pending_out[s] is not None:
      pending_out[s].wait()


def kernel(feats, domain_idx, gamma, beta):
  n, d = feats.shape
  assert n % (NW * CHUNK) == 0 and d % L == 0
  rows_per_worker = n // NW
  n_chunks = rows_per_worker // CHUNK

  idx32 = domain_idx.astype(jnp.int32)

  mesh = plsc.VectorSubcoreMesh(core_axis_name="c", subcore_axis_name="s")
  body = functools.partial(
      _film_body, rows_per_worker=rows_per_worker, n_chunks=n_chunks, d=d)
  return pl.kernel(
      body,
      out_type=jax.ShapeDtypeStruct((n, d), jnp.float32),
      mesh=mesh,
      scratch_types=[
          pltpu.VMEM((n_chunks, CHUNK), jnp.int32),
          pltpu.VMEM((CHUNK, d), jnp.float32),
          pltpu.VMEM((CHUNK, d), jnp.float32),
          pltpu.VMEM((CHUNK, d), jnp.float32),
          pltpu.VMEM((CHUNK, d), jnp.float32),
          pltpu.VMEM((CHUNK, d), jnp.float32),
          pltpu.VMEM((CHUNK, d), jnp.float32),
          pltpu.SemaphoreType.DMA,
          pltpu.SemaphoreType.DMA,
          pltpu.SemaphoreType.DMA,
          pltpu.SemaphoreType.DMA,
          pltpu.SemaphoreType.DMA,
      ],
  )(feats, idx32, gamma, beta)


# speedup vs baseline: 3.2545x; 1.0416x over previous
import functools

import jax
import jax.numpy as jnp
from jax import lax
from jax.experimental import pallas as pl
from jax.experimental.pallas import tpu as pltpu
from jax.experimental.pallas import tpu_sc as plsc

L = 16
NUM_CORES = 2
NUM_SUBCORES = 16
NW = NUM_CORES * NUM_SUBCORES

CHUNK = 64
SLOTS = 6
RUNROLL = 2


def _film_body(feats_hbm, idx_hbm, gamma_hbm, beta_hbm, out_hbm,
               idx_v, g_v, f_v, sem_idx, sem_in, sem_add, sem_out,
               *, rows_per_worker, n_chunks, d):
  wid = lax.axis_index("s") * NUM_CORES + lax.axis_index("c")
  base = wid * rows_per_worker

  idx_cps = [
      pltpu.async_copy(idx_hbm.at[pl.ds(base + k * CHUNK, CHUNK)],
                       idx_v.at[k], sem_idx)
      for k in range(n_chunks)
  ]
  for cp in idx_cps:
    cp.wait()

  pending_in = [None] * SLOTS
  pending_add = [None] * SLOTS
  pending_out = [None] * SLOTS

  def start_in(k):
    s = k % SLOTS
    if pending_add[s] is not None:
      pending_add[s].wait()
    if pending_out[s] is not None:
      pending_out[s].wait()
    pending_in[s] = [
        pltpu.async_copy(gamma_hbm.at[idx_v.at[k]], g_v.at[s], sem_in[s]),
        pltpu.async_copy(feats_hbm.at[pl.ds(base + k * CHUNK, CHUNK)],
                         f_v.at[s], sem_in[s]),
    ]

  def compute(s):
    g = g_v.at[s]
    f = f_v.at[s]

    def row_body(r0, rcarry):
      for u in range(RUNROLL):
        r = r0 * RUNROLL + u
        for j in range(d // L):
          sl = pl.ds(j * L, L)
          f[r, sl] = f[r, sl] + f[r, sl] * g[r, sl]
      return rcarry

    lax.fori_loop(0, CHUNK // RUNROLL, row_body, 0)

  start_in(0)
  start_in(1)
  for k in range(n_chunks):
    s = k % SLOTS
    for cp in pending_in[s]:
      cp.wait()
    compute(s)
    pending_add[s] = pltpu.async_copy(
        beta_hbm.at[idx_v.at[k]], f_v.at[s], sem_add[s], add=True)
    if k >= 1:
      ps = (k - 1) % SLOTS
      pending_add[ps].wait()
      pending_add[ps] = None
      pending_out[ps] = pltpu.async_copy(
          f_v.at[ps], out_hbm.at[pl.ds(base + (k - 1) * CHUNK, CHUNK)],
          sem_out[ps])
    if k + 2 < n_chunks:
      start_in(k + 2)
  ls = (n_chunks - 1) % SLOTS
  pending_add[ls].wait()
  pending_out[ls] = pltpu.async_copy(
      f_v.at[ls], out_hbm.at[pl.ds(base + (n_chunks - 1) * CHUNK, CHUNK)],
      sem_out[ls])
  for s in range(SLOTS):
    if pending_out[s] is not None:
      pending_out[s].wait()


def kernel(feats, domain_idx, gamma, beta):
  n, d = feats.shape
  assert n % (NW * CHUNK) == 0 and d % L == 0
  rows_per_worker = n // NW
  n_chunks = rows_per_worker // CHUNK
  assert n_chunks >= 2

  idx32 = domain_idx.astype(jnp.int32)

  mesh = plsc.VectorSubcoreMesh(core_axis_name="c", subcore_axis_name="s")
  body = functools.partial(
      _film_body, rows_per_worker=rows_per_worker, n_chunks=n_chunks, d=d)
  return pl.kernel(
      body,
      out_type=jax.ShapeDtypeStruct((n, d), jnp.float32),
      mesh=mesh,
      scratch_types=[
          pltpu.VMEM((n_chunks, CHUNK), jnp.int32),
          pltpu.VMEM((SLOTS, CHUNK, d), jnp.float32),
          pltpu.VMEM((SLOTS, CHUNK, d), jnp.float32),
          pltpu.SemaphoreType.DMA,
          [pltpu.SemaphoreType.DMA] * SLOTS,
          [pltpu.SemaphoreType.DMA] * SLOTS,
          [pltpu.SemaphoreType.DMA] * SLOTS,
      ],
  )(feats, idx32, gamma, beta)
